# baseline (speedup 1.0000x reference)
import jax
import jax.numpy as jnp
from jax.experimental import pallas as pl
from jax.experimental.pallas import tpu as pltpu

BLOCK_M = 1024
SUB_M = 512


def _router_kernel(x_ref, w_ref, b_ref, o_ref):
    j = pl.program_id(1)
    xs = x_ref[pl.ds(j * SUB_M, SUB_M), :]
    logits_t = jax.lax.dot_general(
        w_ref[...], xs,
        dimension_numbers=(((1,), (1,)), ((), ())),
        preferred_element_type=jnp.float32)
    e = jnp.exp(logits_t + b_ref[...])
    o_ref[...] = e * pl.reciprocal(jnp.sum(e, axis=0, keepdims=True))


def kernel(x, W, b):
    n_tokens, embed_dim = x.shape
    n_experts = W.shape[0]
    b2 = b.reshape(n_experts, 1)
    grid = (n_tokens // BLOCK_M, BLOCK_M // SUB_M)
    probs_t = pl.pallas_call(
        _router_kernel,
        grid=grid,
        in_specs=[
            pl.BlockSpec((BLOCK_M, embed_dim), lambda i, j: (i, 0),
                         pipeline_mode=pl.Buffered(buffer_count=2)),
            pl.BlockSpec((n_experts, embed_dim), lambda i, j: (0, 0)),
            pl.BlockSpec((n_experts, 1), lambda i, j: (0, 0)),
        ],
        out_specs=pl.BlockSpec((n_experts, SUB_M), lambda i, j: (0, 2 * i + j)),
        out_shape=jax.ShapeDtypeStruct((n_experts, n_tokens), jnp.float32),
        compiler_params=pltpu.CompilerParams(
            dimension_semantics=("arbitrary", "arbitrary"),
        ),
    )(x, W, b2)
    return probs_t.T


# speedup vs baseline: 1.6076x; 1.6076x over previous
import jax
import jax.numpy as jnp
from jax.experimental import pallas as pl
from jax.experimental.pallas import tpu as pltpu

BLOCK_M = 1024


def _router_kernel(x_ref, w_ref, b_ref, o_ref):
    logits_t = jax.lax.dot_general(
        w_ref[...], x_ref[...],
        dimension_numbers=(((1,), (1,)), ((), ())),
        preferred_element_type=jnp.float32)
    e = jnp.exp(logits_t + b_ref[...])
    o_ref[...] = e * pl.reciprocal(jnp.sum(e, axis=0, keepdims=True))


def kernel(x, W, b):
    n_tokens, embed_dim = x.shape
    n_experts = W.shape[0]
    b2 = b.reshape(n_experts, 1)
    grid = (n_tokens // BLOCK_M,)
    probs_t = pl.pallas_call(
        _router_kernel,
        grid=grid,
        in_specs=[
            pl.BlockSpec((BLOCK_M, embed_dim), lambda i: (i, 0)),
            pl.BlockSpec((n_experts, embed_dim), lambda i: (0, 0)),
            pl.BlockSpec((n_experts, 1), lambda i: (0, 0)),
        ],
        out_specs=pl.BlockSpec((n_experts, BLOCK_M), lambda i: (0, i)),
        out_shape=jax.ShapeDtypeStruct((n_experts, n_tokens), jnp.float32),
        compiler_params=pltpu.CompilerParams(
            dimension_semantics=("parallel",),
        ),
    )(x, W, b2)
    return probs_t.T
